# Initial kernel scaffold; baseline (speedup 1.0000x reference)
import jax
import jax.numpy as jnp
from jax.experimental import pallas as pl


def kernel(edge_index, atom_type, edge_J, edge_spin_distance):
    raise NotImplementedError("write your pallas kernel here")


# speedup vs baseline: 20.9171x; 20.9171x over previous
import functools
import math

import jax
import jax.numpy as jnp
from jax import lax
from jax.experimental import pallas as pl
from jax.experimental.pallas import tpu as pltpu
from jax.experimental.pallas import tpu_sc as plsc

AVG_NUM_NEIGHBORS = 64.0
FACTOR = 1.0 / math.sqrt(AVG_NUM_NEIGHBORS)

NC = 2
NS = 16
NW = NC * NS
LANES = 16
ROW = 128
ROWS_PER_CHUNK = 80
CHUNK = ROW * ROWS_PER_CHUNK


def _sc_scatter_kernel(E, N):
    assert E % ROW == 0
    e_rows = E // ROW
    n_chunks = -(-e_rows // ROWS_PER_CHUNK)
    chunks_per_worker = -(-n_chunks // NW)
    nps = -(-N // (NS * 8)) * 8
    n_pad = nps * NS

    mesh = plsc.VectorSubcoreMesh(core_axis_name="c", subcore_axis_name="s")

    @functools.partial(
        pl.kernel,
        out_type=(
            jax.ShapeDtypeStruct((E,), jnp.float32),
            jax.ShapeDtypeStruct((NC * n_pad,), jnp.float32),
        ),
        mesh=mesh,
        scratch_types=dict(
            idx_v=pltpu.VMEM((ROWS_PER_CHUNK, ROW), jnp.int32),
            j_v=pltpu.VMEM((CHUNK,), jnp.float32),
            s_v=pltpu.VMEM((CHUNK,), jnp.float32),
            eng_v=pltpu.VMEM((CHUNK,), jnp.float32),
            stage_v=pltpu.VMEM((nps,), jnp.float32),
            acc_sh=pltpu.VMEM_SHARED((n_pad,), jnp.float32),
        ),
    )
    def body(center_hbm, j_hbm, s_hbm, eng_hbm, partial_hbm,
             idx_v, j_v, s_v, eng_v, stage_v, acc_sh):
        cid = lax.axis_index("c")
        sid = lax.axis_index("s")
        wid = sid * NC + cid

        def zero_body(i, _):
            stage_v[pl.ds(i * LANES, LANES)] = jnp.zeros((LANES,), jnp.float32)
            return 0
        lax.fori_loop(0, nps // LANES, zero_body, 0)
        pltpu.sync_copy(stage_v, acc_sh.at[pl.ds(sid * nps, nps)])
        plsc.subcore_barrier()

        def chunk_body(k, _):
            chunk = wid + k * NW

            @pl.when(chunk < n_chunks)
            def _():
                row_off = chunk * ROWS_PER_CHUNK
                off = chunk * CHUNK
                pltpu.sync_copy(center_hbm.at[pl.ds(row_off, ROWS_PER_CHUNK)],
                                idx_v)
                pltpu.sync_copy(j_hbm.at[pl.ds(off, CHUNK)], j_v)
                pltpu.sync_copy(s_hbm.at[pl.ds(off, CHUNK)], s_v)

                def mul_body(t, _):
                    sl = pl.ds(t * LANES, LANES)
                    eng_v[sl] = j_v[sl] * s_v[sl]
                    return 0
                lax.fori_loop(0, CHUNK // LANES, mul_body, 0, unroll=8)

                pltpu.sync_copy(eng_v, eng_hbm.at[pl.ds(off, CHUNK)])

                def scat_body(r, _):
                    pltpu.sync_copy(eng_v.at[pl.ds(r * ROW, ROW)],
                                    acc_sh.at[idx_v.at[r]], add=True)
                    return 0
                lax.fori_loop(0, ROWS_PER_CHUNK, scat_body, 0)
            return 0
        lax.fori_loop(0, chunks_per_worker, chunk_body, 0)

        plsc.subcore_barrier()
        pltpu.sync_copy(acc_sh.at[pl.ds(sid * nps, nps)], stage_v)
        pltpu.sync_copy(stage_v,
                        partial_hbm.at[pl.ds(cid * n_pad + sid * nps, nps)])

    return body, n_pad


def _combine_kernel(p_ref, o_ref):
    o_ref[...] = (p_ref[0] + p_ref[1]) * FACTOR


def kernel(edge_index, atom_type, edge_J, edge_spin_distance):
    N = atom_type.shape[0]
    E = edge_J.shape[0]
    center2d = edge_index[0].reshape(E // ROW, ROW)
    j_flat = edge_J.reshape(E)

    sc_fn, n_pad = _sc_scatter_kernel(E, N)
    eng_flat, partial = sc_fn(center2d, j_flat, edge_spin_distance)

    p3 = partial.reshape(NC, n_pad // 128, 128)
    atom_pad = pl.pallas_call(
        _combine_kernel,
        out_shape=jax.ShapeDtypeStruct((n_pad // 128, 128), jnp.float32),
    )(p3)
    atom_eng = atom_pad.reshape(n_pad)[:N].reshape(N, 1)
    return eng_flat.reshape(E, 1), atom_eng


# speedup vs baseline: 34.1034x; 1.6304x over previous
import functools
import math

import jax
import jax.numpy as jnp
from jax import lax
from jax.experimental import pallas as pl
from jax.experimental.pallas import tpu as pltpu
from jax.experimental.pallas import tpu_sc as plsc

AVG_NUM_NEIGHBORS = 64.0
FACTOR = 1.0 / math.sqrt(AVG_NUM_NEIGHBORS)

NC = 2
NS = 16
NW = NC * NS
LANES = 16
ROW = 128
ROWS_PER_CHUNK = 80
CHUNK = ROW * ROWS_PER_CHUNK


def _sc_scatter_kernel(E, N):
    assert E % ROW == 0
    e_rows = E // ROW
    n_chunks = -(-e_rows // ROWS_PER_CHUNK)
    chunks_per_worker = -(-n_chunks // NW)
    nps = -(-N // (NS * 8)) * 8
    n_pad = nps * NS

    mesh = plsc.VectorSubcoreMesh(core_axis_name="c", subcore_axis_name="s")

    @functools.partial(
        pl.kernel,
        out_type=(
            jax.ShapeDtypeStruct((E,), jnp.float32),
            jax.ShapeDtypeStruct((NC * n_pad,), jnp.float32),
        ),
        mesh=mesh,
        scratch_types=dict(
            idx_v=pltpu.VMEM((ROWS_PER_CHUNK, ROW), jnp.int32),
            j_v=pltpu.VMEM((CHUNK,), jnp.float32),
            s_v=pltpu.VMEM((CHUNK,), jnp.float32),
            eng_v=pltpu.VMEM((CHUNK,), jnp.float32),
            stage_v=pltpu.VMEM((nps,), jnp.float32),
            acc_sh=pltpu.VMEM_SHARED((n_pad,), jnp.float32),
            in_sem=pltpu.SemaphoreType.DMA,
            out_sem=pltpu.SemaphoreType.DMA,
        ),
    )
    def body(center_hbm, j_hbm, s_hbm, eng_hbm, partial_hbm,
             idx_v, j_v, s_v, eng_v, stage_v, acc_sh, in_sem, out_sem):
        cid = lax.axis_index("c")
        sid = lax.axis_index("s")
        wid = sid * NC + cid

        def zero_body(i, _):
            stage_v[pl.ds(i * LANES, LANES)] = jnp.zeros((LANES,), jnp.float32)
            return 0
        lax.fori_loop(0, nps // LANES, zero_body, 0)
        pltpu.sync_copy(stage_v, acc_sh.at[pl.ds(sid * nps, nps)])
        plsc.subcore_barrier()

        def chunk_body(k, _):
            chunk = wid + k * NW

            @pl.when(chunk < n_chunks)
            def _():
                row_off = chunk * ROWS_PER_CHUNK
                off = chunk * CHUNK
                d1 = pltpu.async_copy(
                    center_hbm.at[pl.ds(row_off, ROWS_PER_CHUNK)],
                    idx_v, in_sem)
                d2 = pltpu.async_copy(j_hbm.at[pl.ds(off, CHUNK)], j_v, in_sem)
                d3 = pltpu.async_copy(s_hbm.at[pl.ds(off, CHUNK)], s_v, in_sem)
                d1.wait()
                d2.wait()
                d3.wait()

                def mul_body(t, _):
                    sl = pl.ds(t * LANES, LANES)
                    eng_v[sl] = j_v[sl] * s_v[sl]
                    return 0
                lax.fori_loop(0, CHUNK // LANES, mul_body, 0, unroll=8)

                wb = pltpu.async_copy(eng_v, eng_hbm.at[pl.ds(off, CHUNK)],
                                      in_sem)

                def scat_body(r, _):
                    pltpu.async_copy(eng_v.at[pl.ds(r * ROW, ROW)],
                                     acc_sh.at[idx_v.at[r]], out_sem, add=True)
                    return 0
                lax.fori_loop(0, ROWS_PER_CHUNK, scat_body, 0)

                def drain_body(r, _):
                    pltpu.make_async_copy(
                        eng_v.at[pl.ds(r * ROW, ROW)],
                        acc_sh.at[idx_v.at[r]], out_sem).wait()
                    return 0
                lax.fori_loop(0, ROWS_PER_CHUNK, drain_body, 0)
                wb.wait()
            return 0
        lax.fori_loop(0, chunks_per_worker, chunk_body, 0)

        plsc.subcore_barrier()
        pltpu.sync_copy(acc_sh.at[pl.ds(sid * nps, nps)], stage_v)
        pltpu.sync_copy(stage_v,
                        partial_hbm.at[pl.ds(cid * n_pad + sid * nps, nps)])

    return body, n_pad


def _combine_kernel(p_ref, o_ref):
    o_ref[...] = (p_ref[0] + p_ref[1]) * FACTOR


def kernel(edge_index, atom_type, edge_J, edge_spin_distance):
    N = atom_type.shape[0]
    E = edge_J.shape[0]
    center2d = edge_index[0].reshape(E // ROW, ROW)
    j_flat = edge_J.reshape(E)

    sc_fn, n_pad = _sc_scatter_kernel(E, N)
    eng_flat, partial = sc_fn(center2d, j_flat, edge_spin_distance)

    p3 = partial.reshape(NC, n_pad // 128, 128)
    atom_pad = pl.pallas_call(
        _combine_kernel,
        out_shape=jax.ShapeDtypeStruct((n_pad // 128, 128), jnp.float32),
    )(p3)
    atom_eng = atom_pad.reshape(n_pad)[:N].reshape(N, 1)
    return eng_flat.reshape(E, 1), atom_eng
